# baseline (speedup 1.0000x reference)
import functools

import jax
import jax.numpy as jnp
from jax import lax
from jax.experimental import pallas as pl
from jax.experimental.pallas import tpu as pltpu
from jax.experimental.pallas import tpu_sc as plsc

N_NODES = 10000
N_EDGES = 320000
F_HID = 128
N_CLS = 64

NC = 2
NS = 16
EDGES_PER_SC = N_EDGES // NC
EDGES_PER_TILE = EDGES_PER_SC // NS
CHUNK = 40
HALF = 4
SLOTS = 2 * HALF
N_CHUNKS = EDGES_PER_TILE // CHUNK
N_ITER = N_CHUNKS // SLOTS
TAIL0 = N_ITER * SLOTS

DCHUNK = 125
DN_CHUNKS = EDGES_PER_TILE // DCHUNK
DK = 8
DN_ITER = DN_CHUNKS // (2 * DK)
DTAIL0 = DN_ITER * 2 * DK
ZROWS = 624
ZREM = N_NODES - NS * ZROWS

@functools.cache
def _sc_kernels():
    mesh = plsc.VectorSubcoreMesh(
        core_axis_name="c", subcore_axis_name="s", num_cores=NC, num_subcores=NS
    )

    @functools.partial(
        pl.kernel,
        mesh=mesh,
        out_type=jax.ShapeDtypeStruct((NC * N_NODES,), jnp.float32),
        scratch_types=[
            pltpu.VMEM_SHARED((N_NODES,), jnp.float32),
            pltpu.VMEM((2, DK, DCHUNK), jnp.int32),
            pltpu.VMEM((128,), jnp.float32),
            pltpu.VMEM((N_NODES,), jnp.float32),
            pltpu.SemaphoreType.DMA,
            pltpu.SemaphoreType.DMA,
        ],
    )
    def _sc_degree(dstr_hbm, out_hbm, acc_sh, dst_v, ones_v, stage_v,
                   sem_i, sem_s):
        c = lax.axis_index("c")
        s = lax.axis_index("s")

        @pl.when(s == 0)
        def _zero():
            def zbody(i, carry):
                stage_v[pl.ds(i * 16, 16)] = jnp.zeros((16,), jnp.float32)
                return carry
            lax.fori_loop(0, N_NODES // 16, zbody, 0)
            pltpu.sync_copy(stage_v, acc_sh)

        for k in range(128 // 16):
            ones_v[pl.ds(k * 16, 16)] = jnp.ones((16,), jnp.float32)
        plsc.subcore_barrier()

        tile_g0 = (c * EDGES_PER_SC + s * EDGES_PER_TILE) // DCHUNK

        def fire_idxd(half, gbase):
            pltpu.async_copy(
                dstr_hbm.at[pl.ds(pl.multiple_of(tile_g0 + gbase, 8), DK)],
                dst_v.at[half], sem_i)

        def wait_idxd(half):
            pltpu.make_async_copy(dstr_hbm.at[pl.ds(0, DK)], dst_v.at[half],
                                  sem_i).wait()

        def fire_scatd(half):
            for j in range(DK):
                pltpu.async_copy(ones_v.at[pl.ds(0, DCHUNK)],
                                 acc_sh.at[dst_v.at[half, j]], sem_s,
                                 add=True)

        def wait_scatd():
            for j in range(DK):
                pltpu.make_async_copy(out_hbm.at[pl.ds(0, DCHUNK)],
                                      ones_v.at[pl.ds(0, DCHUNK)],
                                      sem_s).wait()

        fire_idxd(0, 0)
        wait_idxd(0)
        fire_scatd(0)
        fire_idxd(1, DK)
        wait_scatd()
        wait_idxd(1)
        fire_scatd(1)
        fire_idxd(0, 2 * DK)

        def dsteady(i, carry):
            gbase = i * 2 * DK
            wait_scatd()
            wait_idxd(0)
            fire_scatd(0)
            fire_idxd(1, gbase + DK)
            wait_scatd()
            wait_idxd(1)
            fire_scatd(1)

            @pl.when(i < DN_ITER - 1)
            def _prefetch():
                fire_idxd(0, gbase + 2 * DK)

            return carry

        lax.fori_loop(1, DN_ITER, dsteady, 0)
        wait_scatd()

        for t in range(DTAIL0, DN_CHUNKS):
            pltpu.async_copy(dstr_hbm.at[pl.ds(tile_g0 + t, 1)],
                             dst_v.at[0].at[pl.ds(0, 1)], sem_i).wait()
            pltpu.async_copy(ones_v.at[pl.ds(0, DCHUNK)],
                             acc_sh.at[dst_v.at[0, 0]], sem_s,
                             add=True).wait()
        plsc.subcore_barrier()

        @pl.when(s == 0)
        def _copy_out():
            pltpu.sync_copy(acc_sh, stage_v)
            pltpu.sync_copy(
                stage_v,
                out_hbm.at[pl.ds(pl.multiple_of(c * N_NODES, 8), N_NODES)])

    @functools.partial(
        pl.kernel,
        mesh=mesh,
        out_type=jax.ShapeDtypeStruct((NC * N_NODES, F_HID), jnp.float32),
        scratch_types=[
            pltpu.VMEM_SHARED((N_NODES, F_HID), jnp.float32),
            pltpu.VMEM((2, HALF, 2, CHUNK), jnp.int32),
            pltpu.VMEM((SLOTS, CHUNK, F_HID), jnp.float32),
            pltpu.SemaphoreType.DMA,
            pltpu.SemaphoreType.DMA,
            pltpu.SemaphoreType.DMA,
        ],
    )
    def _sc_scatter(z_hbm, epack_hbm, zeros_hbm, out_hbm,
                    acc_sh, idx_v, rows_v, sem_i, sem_g, sem_s):
        c = lax.axis_index("c")
        s = lax.axis_index("s")
        row0 = pl.multiple_of(s * ZROWS, 8)

        pltpu.sync_copy(zeros_hbm, acc_sh.at[pl.ds(row0, ZROWS)])

        @pl.when(s == 0)
        def _zero_rem():
            pltpu.sync_copy(zeros_hbm.at[pl.ds(0, ZREM)],
                            acc_sh.at[pl.ds(NS * ZROWS, ZREM)])

        plsc.subcore_barrier()

        tile_c0 = c * (EDGES_PER_SC // CHUNK) + s * N_CHUNKS

        def fire_idx(half, cbase):
            pltpu.async_copy(
                epack_hbm.at[pl.ds(tile_c0 + cbase, HALF)],
                idx_v.at[half], sem_i)

        def wait_idx(half):
            pltpu.make_async_copy(epack_hbm.at[pl.ds(0, HALF)],
                                  idx_v.at[half], sem_i).wait()

        def fire_gather(half):
            for j in range(HALF):
                slot = half * HALF + j
                pltpu.async_copy(z_hbm.at[idx_v.at[half, j, 0]],
                                 rows_v.at[slot], sem_g)

        def fire_scatter(half):
            for j in range(HALF):
                slot = half * HALF + j
                pltpu.async_copy(rows_v.at[slot],
                                 acc_sh.at[idx_v.at[half, j, 1]],
                                 sem_s, add=True)

        def wait_rows(half, sem):
            for j in range(HALF):
                slot = half * HALF + j
                pltpu.make_async_copy(z_hbm.at[pl.ds(0, CHUNK)],
                                      rows_v.at[slot], sem).wait()

        fire_idx(0, 0)
        wait_idx(0)
        fire_gather(0)
        fire_idx(1, HALF)
        wait_rows(0, sem_g)
        fire_scatter(0)
        wait_idx(1)
        fire_gather(1)
        wait_rows(0, sem_s)
        fire_idx(0, SLOTS)
        wait_rows(1, sem_g)
        fire_scatter(1)
        wait_idx(0)
        fire_gather(0)

        def steady_body(i, carry):
            c0 = i * SLOTS
            wait_rows(1, sem_s)
            fire_idx(1, c0 + HALF)
            wait_rows(0, sem_g)
            fire_scatter(0)
            wait_idx(1)
            fire_gather(1)
            wait_rows(0, sem_s)

            @pl.when(i < N_ITER - 1)
            def _prefetch_idx():
                fire_idx(0, c0 + SLOTS)

            wait_rows(1, sem_g)
            fire_scatter(1)

            @pl.when(i < N_ITER - 1)
            def _prefetch_gather():
                wait_idx(0)
                fire_gather(0)

            return carry

        lax.fori_loop(1, N_ITER, steady_body, 0)
        wait_rows(1, sem_s)

        for t in range(TAIL0, N_CHUNKS):
            pltpu.async_copy(
                epack_hbm.at[pl.ds(tile_c0 + t, 1)],
                idx_v.at[0].at[pl.ds(0, 1)], sem_i).wait()
            pltpu.async_copy(z_hbm.at[idx_v.at[0, 0, 0]], rows_v.at[0],
                             sem_g).wait()
            pltpu.async_copy(rows_v.at[0], acc_sh.at[idx_v.at[0, 0, 1]],
                             sem_s, add=True).wait()
        plsc.subcore_barrier()

        out_off = pl.multiple_of(c * N_NODES + row0, 8)
        pltpu.sync_copy(acc_sh.at[pl.ds(row0, ZROWS)],
                        out_hbm.at[pl.ds(out_off, ZROWS)])

        @pl.when(s == 0)
        def _out_rem():
            rem0 = NS * ZROWS
            pltpu.sync_copy(
                acc_sh.at[pl.ds(rem0, ZREM)],
                out_hbm.at[pl.ds(pl.multiple_of(c * N_NODES + rem0, 8), ZREM)])

    return _sc_degree, _sc_scatter


BLK = 1000


def _tc_first_body(d0, d1, x, w, dinv_ref, z_ref):
    dinv = lax.rsqrt(d0[...] + d1[...] + 1.0)
    dinv_ref[...] = dinv
    z_ref[...] = dinv * jnp.dot(x[...], w[...], preferred_element_type=jnp.float32)


def _tc_mid_body(a0, a1, z, dinv, b, w, zn_ref):
    h = jnp.maximum(dinv[...] * (a0[...] + a1[...] + z[...]) + b[...], 0.0)
    zn_ref[...] = dinv[...] * jnp.dot(h, w[...], preferred_element_type=jnp.float32)


def _tc_final_body(a0, a1, z, dinv, b3, wl, bl, out_ref):
    x3 = dinv[...] * (a0[...] + a1[...] + z[...]) + b3[...]
    logits = jnp.dot(x3, wl[...], preferred_element_type=jnp.float32) + bl[...]
    m = jnp.max(logits, axis=1, keepdims=True)
    lse = jnp.log(jnp.sum(jnp.exp(logits - m), axis=1, keepdims=True))
    out_ref[...] = (logits - m) - lse


def _rows(i):
    return (i, 0)


def _whole(i):
    return (0, 0)


_GRID = N_NODES // BLK

_tc_first = pl.pallas_call(
    _tc_first_body,
    grid=(_GRID,),
    in_specs=[
        pl.BlockSpec((BLK, 1), _rows),
        pl.BlockSpec((BLK, 1), _rows),
        pl.BlockSpec((BLK, F_HID), _rows),
        pl.BlockSpec((F_HID, F_HID), _whole),
    ],
    out_specs=[
        pl.BlockSpec((BLK, 1), _rows),
        pl.BlockSpec((BLK, F_HID), _rows),
    ],
    out_shape=[
        jax.ShapeDtypeStruct((N_NODES, 1), jnp.float32),
        jax.ShapeDtypeStruct((N_NODES, F_HID), jnp.float32),
    ],
)

_tc_mid = pl.pallas_call(
    _tc_mid_body,
    grid=(_GRID,),
    in_specs=[
        pl.BlockSpec((BLK, F_HID), _rows),
        pl.BlockSpec((BLK, F_HID), _rows),
        pl.BlockSpec((BLK, F_HID), _rows),
        pl.BlockSpec((BLK, 1), _rows),
        pl.BlockSpec((1, F_HID), _whole),
        pl.BlockSpec((F_HID, F_HID), _whole),
    ],
    out_specs=pl.BlockSpec((BLK, F_HID), _rows),
    out_shape=jax.ShapeDtypeStruct((N_NODES, F_HID), jnp.float32),
)

_tc_final = pl.pallas_call(
    _tc_final_body,
    grid=(_GRID,),
    in_specs=[
        pl.BlockSpec((BLK, F_HID), _rows),
        pl.BlockSpec((BLK, F_HID), _rows),
        pl.BlockSpec((BLK, F_HID), _rows),
        pl.BlockSpec((BLK, 1), _rows),
        pl.BlockSpec((1, F_HID), _whole),
        pl.BlockSpec((F_HID, N_CLS), _whole),
        pl.BlockSpec((1, N_CLS), _whole),
    ],
    out_specs=pl.BlockSpec((BLK, N_CLS), _rows),
    out_shape=jax.ShapeDtypeStruct((N_NODES, N_CLS), jnp.float32),
)


def kernel(x, edge_index, W1, b1, W2, b2, W3, b3, Wl, bl):
    dst = edge_index[1]
    epack = edge_index.reshape(2, N_EDGES // CHUNK, CHUNK).transpose(1, 0, 2)
    zeros_blk = jnp.zeros((ZROWS, F_HID), jnp.float32)

    _sc_degree, _sc_scatter = _sc_kernels()
    deg = _sc_degree(dst.reshape(N_EDGES // DCHUNK, DCHUNK)).reshape(NC, N_NODES)
    d0 = deg[0].reshape(N_NODES, 1)
    d1 = deg[1].reshape(N_NODES, 1)

    dinv, z1 = _tc_first(d0, d1, x, W1)
    acc = _sc_scatter(z1, epack, zeros_blk).reshape(NC, N_NODES, F_HID)
    z2 = _tc_mid(acc[0], acc[1], z1, dinv, b1.reshape(1, F_HID), W2)
    acc = _sc_scatter(z2, epack, zeros_blk).reshape(NC, N_NODES, F_HID)
    z3 = _tc_mid(acc[0], acc[1], z2, dinv, b2.reshape(1, F_HID), W3)
    acc = _sc_scatter(z3, epack, zeros_blk).reshape(NC, N_NODES, F_HID)
    return _tc_final(acc[0], acc[1], z3, dinv, b3.reshape(1, F_HID),
                     Wl, bl.reshape(1, N_CLS))


# speedup vs baseline: 1.0205x; 1.0205x over previous
import functools

import jax
import jax.numpy as jnp
from jax import lax
from jax.experimental import pallas as pl
from jax.experimental.pallas import tpu as pltpu
from jax.experimental.pallas import tpu_sc as plsc

N_NODES = 10000
N_EDGES = 320000
F_HID = 128
N_CLS = 64

NC = 2
NS = 16
EDGES_PER_SC = N_EDGES // NC
EDGES_PER_TILE = EDGES_PER_SC // NS
CHUNK = 40
HALF = 4
SLOTS = 2 * HALF
N_CHUNKS = EDGES_PER_TILE // CHUNK
N_ITER = N_CHUNKS // SLOTS
TAIL0 = N_ITER * SLOTS

DCHUNK = 80
DN_CHUNKS = EDGES_PER_TILE // DCHUNK
DK = 4
DN_GROUPS = DN_CHUNKS // DK
DTAIL0 = DN_GROUPS * DK
ZROWS = 624
ZREM = N_NODES - NS * ZROWS

@functools.cache
def _sc_kernels():
    mesh = plsc.VectorSubcoreMesh(
        core_axis_name="c", subcore_axis_name="s", num_cores=NC, num_subcores=NS
    )

    @functools.partial(
        pl.kernel,
        mesh=mesh,
        out_type=jax.ShapeDtypeStruct((NC * N_NODES,), jnp.float32),
        scratch_types=[
            pltpu.VMEM_SHARED((N_NODES,), jnp.float32),
            pltpu.VMEM((DK, DCHUNK), jnp.int32),
            pltpu.VMEM((DCHUNK,), jnp.float32),
            pltpu.VMEM((N_NODES,), jnp.float32),
            pltpu.SemaphoreType.DMA,
            pltpu.SemaphoreType.DMA,
        ],
    )
    def _sc_degree(dst_hbm, out_hbm, acc_sh, dst_v, ones_v, stage_v,
                   sem_i, sem_s):
        c = lax.axis_index("c")
        s = lax.axis_index("s")

        @pl.when(s == 0)
        def _zero():
            def zbody(i, carry):
                stage_v[pl.ds(i * 16, 16)] = jnp.zeros((16,), jnp.float32)
                return carry
            lax.fori_loop(0, N_NODES // 16, zbody, 0)
            pltpu.sync_copy(stage_v, acc_sh)

        for k in range(DCHUNK // 16):
            ones_v[pl.ds(k * 16, 16)] = jnp.ones((16,), jnp.float32)
        plsc.subcore_barrier()

        base = c * EDGES_PER_SC + s * EDGES_PER_TILE

        def group_body(g, carry):
            descs = []
            for j in range(DK):
                off = pl.multiple_of(base + (g * DK + j) * DCHUNK, 8)
                descs.append(pltpu.async_copy(
                    dst_hbm.at[pl.ds(off, DCHUNK)], dst_v.at[j], sem_i))
            for d in descs:
                d.wait()
            sds = [pltpu.async_copy(ones_v, acc_sh.at[dst_v.at[j]], sem_s,
                                    add=True)
                   for j in range(DK)]
            for d in sds:
                d.wait()
            return carry

        lax.fori_loop(0, DN_GROUPS, group_body, 0)
        for t in range(DTAIL0, DN_CHUNKS):
            off = pl.multiple_of(base + t * DCHUNK, 8)
            pltpu.async_copy(dst_hbm.at[pl.ds(off, DCHUNK)], dst_v.at[0],
                             sem_i).wait()
            pltpu.async_copy(ones_v, acc_sh.at[dst_v.at[0]], sem_s,
                             add=True).wait()
        plsc.subcore_barrier()

        @pl.when(s == 0)
        def _copy_out():
            pltpu.sync_copy(acc_sh, stage_v)
            pltpu.sync_copy(
                stage_v,
                out_hbm.at[pl.ds(pl.multiple_of(c * N_NODES, 8), N_NODES)])

    @functools.partial(
        pl.kernel,
        mesh=mesh,
        out_type=jax.ShapeDtypeStruct((NC * N_NODES, F_HID), jnp.float32),
        scratch_types=[
            pltpu.VMEM_SHARED((N_NODES, F_HID), jnp.float32),
            pltpu.VMEM((SLOTS, CHUNK), jnp.int32),
            pltpu.VMEM((SLOTS, CHUNK), jnp.int32),
            pltpu.VMEM((SLOTS, CHUNK, F_HID), jnp.float32),
            pltpu.SemaphoreType.DMA,
            pltpu.SemaphoreType.DMA,
            pltpu.SemaphoreType.DMA,
        ],
    )
    def _sc_scatter(z_hbm, src_hbm, dst_hbm, zeros_hbm, out_hbm,
                    acc_sh, src_v, dst_v, rows_v, sem_i, sem_g, sem_s):
        c = lax.axis_index("c")
        s = lax.axis_index("s")
        row0 = pl.multiple_of(s * ZROWS, 8)

        pltpu.sync_copy(zeros_hbm, acc_sh.at[pl.ds(row0, ZROWS)])

        @pl.when(s == 0)
        def _zero_rem():
            pltpu.sync_copy(zeros_hbm.at[pl.ds(0, ZREM)],
                            acc_sh.at[pl.ds(NS * ZROWS, ZREM)])

        plsc.subcore_barrier()

        base = c * EDGES_PER_SC + s * EDGES_PER_TILE

        def fire_idx(half, cbase):
            for j in range(HALF):
                off = pl.multiple_of(base + (cbase + j) * CHUNK, 8)
                slot = half * HALF + j
                pltpu.async_copy(src_hbm.at[pl.ds(off, CHUNK)], src_v.at[slot],
                                 sem_i)
                pltpu.async_copy(dst_hbm.at[pl.ds(off, CHUNK)], dst_v.at[slot],
                                 sem_i)

        def wait_idx(half):
            for j in range(HALF):
                slot = half * HALF + j
                pltpu.make_async_copy(src_hbm.at[pl.ds(0, CHUNK)],
                                      src_v.at[slot], sem_i).wait()
                pltpu.make_async_copy(src_hbm.at[pl.ds(0, CHUNK)],
                                      dst_v.at[slot], sem_i).wait()

        def fire_gather(half):
            for j in range(HALF):
                slot = half * HALF + j
                pltpu.async_copy(z_hbm.at[src_v.at[slot]], rows_v.at[slot],
                                 sem_g)

        def fire_scatter(half):
            for j in range(HALF):
                slot = half * HALF + j
                pltpu.async_copy(rows_v.at[slot], acc_sh.at[dst_v.at[slot]],
                                 sem_s, add=True)

        def wait_rows(half, sem):
            for j in range(HALF):
                slot = half * HALF + j
                pltpu.make_async_copy(z_hbm.at[pl.ds(0, CHUNK)],
                                      rows_v.at[slot], sem).wait()

        fire_idx(0, 0)
        wait_idx(0)
        fire_gather(0)
        fire_idx(1, HALF)
        wait_rows(0, sem_g)
        fire_scatter(0)
        wait_idx(1)
        fire_gather(1)
        wait_rows(0, sem_s)
        fire_idx(0, SLOTS)
        wait_rows(1, sem_g)
        fire_scatter(1)
        wait_idx(0)
        fire_gather(0)

        def steady_body(i, carry):
            c0 = i * SLOTS
            wait_rows(1, sem_s)
            fire_idx(1, c0 + HALF)
            wait_rows(0, sem_g)
            fire_scatter(0)
            wait_idx(1)
            fire_gather(1)
            wait_rows(0, sem_s)

            @pl.when(i < N_ITER - 1)
            def _prefetch_idx():
                fire_idx(0, c0 + SLOTS)

            wait_rows(1, sem_g)
            fire_scatter(1)

            @pl.when(i < N_ITER - 1)
            def _prefetch_gather():
                wait_idx(0)
                fire_gather(0)

            return carry

        lax.fori_loop(1, N_ITER, steady_body, 0)
        wait_rows(1, sem_s)

        for t in range(TAIL0, N_CHUNKS):
            off = pl.multiple_of(base + t * CHUNK, 8)
            pltpu.async_copy(src_hbm.at[pl.ds(off, CHUNK)], src_v.at[0],
                             sem_i).wait()
            pltpu.async_copy(dst_hbm.at[pl.ds(off, CHUNK)], dst_v.at[0],
                             sem_i).wait()
            pltpu.async_copy(z_hbm.at[src_v.at[0]], rows_v.at[0], sem_g).wait()
            pltpu.async_copy(rows_v.at[0], acc_sh.at[dst_v.at[0]], sem_s,
                             add=True).wait()
        plsc.subcore_barrier()

        out_off = pl.multiple_of(c * N_NODES + row0, 8)
        pltpu.sync_copy(acc_sh.at[pl.ds(row0, ZROWS)],
                        out_hbm.at[pl.ds(out_off, ZROWS)])

        @pl.when(s == 0)
        def _out_rem():
            rem0 = NS * ZROWS
            pltpu.sync_copy(
                acc_sh.at[pl.ds(rem0, ZREM)],
                out_hbm.at[pl.ds(pl.multiple_of(c * N_NODES + rem0, 8), ZREM)])

    return _sc_degree, _sc_scatter


BLK = 1000


def _tc_first_body(d0, d1, x, w, dinv_ref, z_ref):
    dinv = lax.rsqrt(d0[...] + d1[...] + 1.0)
    dinv_ref[...] = dinv
    z_ref[...] = dinv * jnp.dot(x[...], w[...], preferred_element_type=jnp.float32)


def _tc_mid_body(a0, a1, z, dinv, b, w, zn_ref):
    h = jnp.maximum(dinv[...] * (a0[...] + a1[...] + z[...]) + b[...], 0.0)
    zn_ref[...] = dinv[...] * jnp.dot(h, w[...], preferred_element_type=jnp.float32)


def _tc_final_body(a0, a1, z, dinv, b3, wl, bl, out_ref):
    x3 = dinv[...] * (a0[...] + a1[...] + z[...]) + b3[...]
    logits = jnp.dot(x3, wl[...], preferred_element_type=jnp.float32) + bl[...]
    m = jnp.max(logits, axis=1, keepdims=True)
    lse = jnp.log(jnp.sum(jnp.exp(logits - m), axis=1, keepdims=True))
    out_ref[...] = (logits - m) - lse


def _rows(i):
    return (i, 0)


def _whole(i):
    return (0, 0)


_GRID = N_NODES // BLK

_tc_first = pl.pallas_call(
    _tc_first_body,
    grid=(_GRID,),
    in_specs=[
        pl.BlockSpec((BLK, 1), _rows),
        pl.BlockSpec((BLK, 1), _rows),
        pl.BlockSpec((BLK, F_HID), _rows),
        pl.BlockSpec((F_HID, F_HID), _whole),
    ],
    out_specs=[
        pl.BlockSpec((BLK, 1), _rows),
        pl.BlockSpec((BLK, F_HID), _rows),
    ],
    out_shape=[
        jax.ShapeDtypeStruct((N_NODES, 1), jnp.float32),
        jax.ShapeDtypeStruct((N_NODES, F_HID), jnp.float32),
    ],
)

_tc_mid = pl.pallas_call(
    _tc_mid_body,
    grid=(_GRID,),
    in_specs=[
        pl.BlockSpec((BLK, F_HID), _rows),
        pl.BlockSpec((BLK, F_HID), _rows),
        pl.BlockSpec((BLK, F_HID), _rows),
        pl.BlockSpec((BLK, 1), _rows),
        pl.BlockSpec((1, F_HID), _whole),
        pl.BlockSpec((F_HID, F_HID), _whole),
    ],
    out_specs=pl.BlockSpec((BLK, F_HID), _rows),
    out_shape=jax.ShapeDtypeStruct((N_NODES, F_HID), jnp.float32),
)

_tc_final = pl.pallas_call(
    _tc_final_body,
    grid=(_GRID,),
    in_specs=[
        pl.BlockSpec((BLK, F_HID), _rows),
        pl.BlockSpec((BLK, F_HID), _rows),
        pl.BlockSpec((BLK, F_HID), _rows),
        pl.BlockSpec((BLK, 1), _rows),
        pl.BlockSpec((1, F_HID), _whole),
        pl.BlockSpec((F_HID, N_CLS), _whole),
        pl.BlockSpec((1, N_CLS), _whole),
    ],
    out_specs=pl.BlockSpec((BLK, N_CLS), _rows),
    out_shape=jax.ShapeDtypeStruct((N_NODES, N_CLS), jnp.float32),
)


def kernel(x, edge_index, W1, b1, W2, b2, W3, b3, Wl, bl):
    src = edge_index[0]
    dst = edge_index[1]
    zeros_blk = jnp.zeros((ZROWS, F_HID), jnp.float32)

    _sc_degree, _sc_scatter = _sc_kernels()
    deg = _sc_degree(dst).reshape(NC, N_NODES)
    d0 = deg[0].reshape(N_NODES, 1)
    d1 = deg[1].reshape(N_NODES, 1)

    dinv, z1 = _tc_first(d0, d1, x, W1)
    acc = _sc_scatter(z1, src, dst, zeros_blk).reshape(NC, N_NODES, F_HID)
    z2 = _tc_mid(acc[0], acc[1], z1, dinv, b1.reshape(1, F_HID), W2)
    acc = _sc_scatter(z2, src, dst, zeros_blk).reshape(NC, N_NODES, F_HID)
    z3 = _tc_mid(acc[0], acc[1], z2, dinv, b2.reshape(1, F_HID), W3)
    acc = _sc_scatter(z3, src, dst, zeros_blk).reshape(NC, N_NODES, F_HID)
    return _tc_final(acc[0], acc[1], z3, dinv, b3.reshape(1, F_HID),
                     Wl, bl.reshape(1, N_CLS))


# speedup vs baseline: 1.0258x; 1.0052x over previous
import functools

import jax
import jax.numpy as jnp
from jax import lax
from jax.experimental import pallas as pl
from jax.experimental.pallas import tpu as pltpu
from jax.experimental.pallas import tpu_sc as plsc

N_NODES = 10000
N_EDGES = 320000
F_HID = 128
N_CLS = 64

NC = 2
NS = 16
EDGES_PER_SC = N_EDGES // NC
EDGES_PER_TILE = EDGES_PER_SC // NS
CHUNK = 40
HALF = 4
SLOTS = 2 * HALF
N_CHUNKS = EDGES_PER_TILE // CHUNK
N_ITER = N_CHUNKS // SLOTS
TAIL0 = N_ITER * SLOTS

DCHUNK = 80
DN_CHUNKS = EDGES_PER_TILE // DCHUNK
DK = 4
DN_GROUPS = DN_CHUNKS // DK
DTAIL0 = DN_GROUPS * DK
ZROWS = 624
ZREM = N_NODES - NS * ZROWS

@functools.cache
def _sc_kernels():
    mesh = plsc.VectorSubcoreMesh(
        core_axis_name="c", subcore_axis_name="s", num_cores=NC, num_subcores=NS
    )

    @functools.partial(
        pl.kernel,
        mesh=mesh,
        out_type=jax.ShapeDtypeStruct((NC * N_NODES,), jnp.float32),
        scratch_types=[
            pltpu.VMEM_SHARED((N_NODES,), jnp.float32),
            pltpu.VMEM((DK, DCHUNK), jnp.int32),
            pltpu.VMEM((DCHUNK,), jnp.float32),
            pltpu.VMEM((N_NODES,), jnp.float32),
            pltpu.SemaphoreType.DMA,
            pltpu.SemaphoreType.DMA,
        ],
    )
    def _sc_degree(dst_hbm, out_hbm, acc_sh, dst_v, ones_v, stage_v,
                   sem_i, sem_s):
        c = lax.axis_index("c")
        s = lax.axis_index("s")

        @pl.when(s == 0)
        def _zero():
            def zbody(i, carry):
                stage_v[pl.ds(i * 16, 16)] = jnp.zeros((16,), jnp.float32)
                return carry
            lax.fori_loop(0, N_NODES // 16, zbody, 0)
            pltpu.sync_copy(stage_v, acc_sh)

        for k in range(DCHUNK // 16):
            ones_v[pl.ds(k * 16, 16)] = jnp.ones((16,), jnp.float32)
        plsc.subcore_barrier()

        base = c * EDGES_PER_SC + s * EDGES_PER_TILE

        def group_body(g, carry):
            descs = []
            for j in range(DK):
                off = pl.multiple_of(base + (g * DK + j) * DCHUNK, 8)
                descs.append(pltpu.async_copy(
                    dst_hbm.at[pl.ds(off, DCHUNK)], dst_v.at[j], sem_i))
            for d in descs:
                d.wait()
            sds = [pltpu.async_copy(ones_v, acc_sh.at[dst_v.at[j]], sem_s,
                                    add=True)
                   for j in range(DK)]
            for d in sds:
                d.wait()
            return carry

        lax.fori_loop(0, DN_GROUPS, group_body, 0)
        for t in range(DTAIL0, DN_CHUNKS):
            off = pl.multiple_of(base + t * DCHUNK, 8)
            pltpu.async_copy(dst_hbm.at[pl.ds(off, DCHUNK)], dst_v.at[0],
                             sem_i).wait()
            pltpu.async_copy(ones_v, acc_sh.at[dst_v.at[0]], sem_s,
                             add=True).wait()
        plsc.subcore_barrier()

        @pl.when(s == 0)
        def _copy_out():
            pltpu.sync_copy(acc_sh, stage_v)
            pltpu.sync_copy(
                stage_v,
                out_hbm.at[pl.ds(pl.multiple_of(c * N_NODES, 8), N_NODES)])

    @functools.partial(
        pl.kernel,
        mesh=mesh,
        out_type=jax.ShapeDtypeStruct((NC * N_NODES, F_HID), jnp.float32),
        scratch_types=[
            pltpu.VMEM_SHARED((N_NODES, F_HID), jnp.float32),
            pltpu.VMEM((SLOTS, CHUNK), jnp.int32),
            pltpu.VMEM((SLOTS, CHUNK), jnp.int32),
            pltpu.VMEM((SLOTS, CHUNK, F_HID), jnp.float32),
            pltpu.SemaphoreType.DMA,
            pltpu.SemaphoreType.DMA,
            pltpu.SemaphoreType.DMA,
        ],
    )
    def _sc_scatter(z_hbm, src_hbm, dst_hbm, zeros_hbm, out_hbm,
                    acc_sh, src_v, dst_v, rows_v, sem_i, sem_g, sem_s):
        c = lax.axis_index("c")
        s = lax.axis_index("s")
        row0 = pl.multiple_of(s * ZROWS, 8)
        base = c * EDGES_PER_SC + s * EDGES_PER_TILE

        def fire_idx(half, cbase):
            for j in range(HALF):
                off = pl.multiple_of(base + (cbase + j) * CHUNK, 8)
                slot = half * HALF + j
                pltpu.async_copy(src_hbm.at[pl.ds(off, CHUNK)], src_v.at[slot],
                                 sem_i)
                pltpu.async_copy(dst_hbm.at[pl.ds(off, CHUNK)], dst_v.at[slot],
                                 sem_i)

        def wait_idx(half):
            for j in range(HALF):
                slot = half * HALF + j
                pltpu.make_async_copy(src_hbm.at[pl.ds(0, CHUNK)],
                                      src_v.at[slot], sem_i).wait()
                pltpu.make_async_copy(src_hbm.at[pl.ds(0, CHUNK)],
                                      dst_v.at[slot], sem_i).wait()

        def fire_gather(half):
            for j in range(HALF):
                slot = half * HALF + j
                pltpu.async_copy(z_hbm.at[src_v.at[slot]], rows_v.at[slot],
                                 sem_g)

        def fire_scatter(half):
            for j in range(HALF):
                slot = half * HALF + j
                pltpu.async_copy(rows_v.at[slot], acc_sh.at[dst_v.at[slot]],
                                 sem_s, add=True)

        def wait_rows(half, sem):
            for j in range(HALF):
                slot = half * HALF + j
                pltpu.make_async_copy(z_hbm.at[pl.ds(0, CHUNK)],
                                      rows_v.at[slot], sem).wait()

        fire_idx(0, 0)
        pltpu.sync_copy(zeros_hbm, acc_sh.at[pl.ds(row0, ZROWS)])

        @pl.when(s == 0)
        def _zero_rem():
            pltpu.sync_copy(zeros_hbm.at[pl.ds(0, ZREM)],
                            acc_sh.at[pl.ds(NS * ZROWS, ZREM)])

        wait_idx(0)
        fire_gather(0)
        fire_idx(1, HALF)
        plsc.subcore_barrier()
        wait_rows(0, sem_g)
        fire_scatter(0)
        wait_idx(1)
        fire_gather(1)
        wait_rows(0, sem_s)
        fire_idx(0, SLOTS)
        wait_rows(1, sem_g)
        fire_scatter(1)
        wait_idx(0)
        fire_gather(0)

        def steady_body(i, carry):
            c0 = i * SLOTS
            wait_rows(1, sem_s)
            fire_idx(1, c0 + HALF)
            wait_rows(0, sem_g)
            fire_scatter(0)
            wait_idx(1)
            fire_gather(1)
            wait_rows(0, sem_s)

            @pl.when(i < N_ITER - 1)
            def _prefetch_idx():
                fire_idx(0, c0 + SLOTS)

            wait_rows(1, sem_g)
            fire_scatter(1)

            @pl.when(i < N_ITER - 1)
            def _prefetch_gather():
                wait_idx(0)
                fire_gather(0)

            return carry

        lax.fori_loop(1, N_ITER, steady_body, 0)
        wait_rows(1, sem_s)

        for t in range(TAIL0, N_CHUNKS):
            off = pl.multiple_of(base + t * CHUNK, 8)
            pltpu.async_copy(src_hbm.at[pl.ds(off, CHUNK)], src_v.at[0],
                             sem_i).wait()
            pltpu.async_copy(dst_hbm.at[pl.ds(off, CHUNK)], dst_v.at[0],
                             sem_i).wait()
            pltpu.async_copy(z_hbm.at[src_v.at[0]], rows_v.at[0], sem_g).wait()
            pltpu.async_copy(rows_v.at[0], acc_sh.at[dst_v.at[0]], sem_s,
                             add=True).wait()
        plsc.subcore_barrier()

        out_off = pl.multiple_of(c * N_NODES + row0, 8)
        pltpu.sync_copy(acc_sh.at[pl.ds(row0, ZROWS)],
                        out_hbm.at[pl.ds(out_off, ZROWS)])

        @pl.when(s == 0)
        def _out_rem():
            rem0 = NS * ZROWS
            pltpu.sync_copy(
                acc_sh.at[pl.ds(rem0, ZREM)],
                out_hbm.at[pl.ds(pl.multiple_of(c * N_NODES + rem0, 8), ZREM)])

    return _sc_degree, _sc_scatter


BLK = 1000


def _tc_first_body(d0, d1, x, w, dinv_ref, z_ref):
    dinv = lax.rsqrt(d0[...] + d1[...] + 1.0)
    dinv_ref[...] = dinv
    z_ref[...] = dinv * jnp.dot(x[...], w[...], preferred_element_type=jnp.float32)


def _tc_mid_body(a0, a1, z, dinv, b, w, zn_ref):
    h = jnp.maximum(dinv[...] * (a0[...] + a1[...] + z[...]) + b[...], 0.0)
    zn_ref[...] = dinv[...] * jnp.dot(h, w[...], preferred_element_type=jnp.float32)


def _tc_final_body(a0, a1, z, dinv, b3, wl, bl, out_ref):
    x3 = dinv[...] * (a0[...] + a1[...] + z[...]) + b3[...]
    logits = jnp.dot(x3, wl[...], preferred_element_type=jnp.float32) + bl[...]
    m = jnp.max(logits, axis=1, keepdims=True)
    lse = jnp.log(jnp.sum(jnp.exp(logits - m), axis=1, keepdims=True))
    out_ref[...] = (logits - m) - lse


def _rows(i):
    return (i, 0)


def _whole(i):
    return (0, 0)


_GRID = N_NODES // BLK

_tc_first = pl.pallas_call(
    _tc_first_body,
    grid=(_GRID,),
    in_specs=[
        pl.BlockSpec((BLK, 1), _rows),
        pl.BlockSpec((BLK, 1), _rows),
        pl.BlockSpec((BLK, F_HID), _rows),
        pl.BlockSpec((F_HID, F_HID), _whole),
    ],
    out_specs=[
        pl.BlockSpec((BLK, 1), _rows),
        pl.BlockSpec((BLK, F_HID), _rows),
    ],
    out_shape=[
        jax.ShapeDtypeStruct((N_NODES, 1), jnp.float32),
        jax.ShapeDtypeStruct((N_NODES, F_HID), jnp.float32),
    ],
)

_tc_mid = pl.pallas_call(
    _tc_mid_body,
    grid=(_GRID,),
    in_specs=[
        pl.BlockSpec((BLK, F_HID), _rows),
        pl.BlockSpec((BLK, F_HID), _rows),
        pl.BlockSpec((BLK, F_HID), _rows),
        pl.BlockSpec((BLK, 1), _rows),
        pl.BlockSpec((1, F_HID), _whole),
        pl.BlockSpec((F_HID, F_HID), _whole),
    ],
    out_specs=pl.BlockSpec((BLK, F_HID), _rows),
    out_shape=jax.ShapeDtypeStruct((N_NODES, F_HID), jnp.float32),
)

_tc_final = pl.pallas_call(
    _tc_final_body,
    grid=(_GRID,),
    in_specs=[
        pl.BlockSpec((BLK, F_HID), _rows),
        pl.BlockSpec((BLK, F_HID), _rows),
        pl.BlockSpec((BLK, F_HID), _rows),
        pl.BlockSpec((BLK, 1), _rows),
        pl.BlockSpec((1, F_HID), _whole),
        pl.BlockSpec((F_HID, N_CLS), _whole),
        pl.BlockSpec((1, N_CLS), _whole),
    ],
    out_specs=pl.BlockSpec((BLK, N_CLS), _rows),
    out_shape=jax.ShapeDtypeStruct((N_NODES, N_CLS), jnp.float32),
)


def kernel(x, edge_index, W1, b1, W2, b2, W3, b3, Wl, bl):
    src = edge_index[0]
    dst = edge_index[1]
    zeros_blk = jnp.zeros((ZROWS, F_HID), jnp.float32)

    _sc_degree, _sc_scatter = _sc_kernels()
    deg = _sc_degree(dst).reshape(NC, N_NODES)
    d0 = deg[0].reshape(N_NODES, 1)
    d1 = deg[1].reshape(N_NODES, 1)

    dinv, z1 = _tc_first(d0, d1, x, W1)
    acc = _sc_scatter(z1, src, dst, zeros_blk).reshape(NC, N_NODES, F_HID)
    z2 = _tc_mid(acc[0], acc[1], z1, dinv, b1.reshape(1, F_HID), W2)
    acc = _sc_scatter(z2, src, dst, zeros_blk).reshape(NC, N_NODES, F_HID)
    z3 = _tc_mid(acc[0], acc[1], z2, dinv, b2.reshape(1, F_HID), W3)
    acc = _sc_scatter(z3, src, dst, zeros_blk).reshape(NC, N_NODES, F_HID)
    return _tc_final(acc[0], acc[1], z3, dinv, b3.reshape(1, F_HID),
                     Wl, bl.reshape(1, N_CLS))


# speedup vs baseline: 1.0441x; 1.0179x over previous
import functools

import jax
import jax.numpy as jnp
from jax import lax
from jax.experimental import pallas as pl
from jax.experimental.pallas import tpu as pltpu
from jax.experimental.pallas import tpu_sc as plsc

N_NODES = 10000
N_EDGES = 320000
F_HID = 128
N_CLS = 64

NC = 2
NS = 16
EDGES_PER_SC = N_EDGES // NC
EDGES_PER_TILE = EDGES_PER_SC // NS
CHUNK = 80
HALF = 2
SLOTS = 2 * HALF
N_CHUNKS = EDGES_PER_TILE // CHUNK
N_ITER = N_CHUNKS // SLOTS
TAIL0 = N_ITER * SLOTS

DCHUNK = 80
DN_CHUNKS = EDGES_PER_TILE // DCHUNK
DK = 4
DN_GROUPS = DN_CHUNKS // DK
DTAIL0 = DN_GROUPS * DK
ZROWS = 624
ZREM = N_NODES - NS * ZROWS

@functools.cache
def _sc_kernels():
    mesh = plsc.VectorSubcoreMesh(
        core_axis_name="c", subcore_axis_name="s", num_cores=NC, num_subcores=NS
    )

    @functools.partial(
        pl.kernel,
        mesh=mesh,
        out_type=jax.ShapeDtypeStruct((NC * N_NODES,), jnp.float32),
        scratch_types=[
            pltpu.VMEM_SHARED((N_NODES,), jnp.float32),
            pltpu.VMEM((DK, DCHUNK), jnp.int32),
            pltpu.VMEM((DCHUNK,), jnp.float32),
            pltpu.VMEM((N_NODES,), jnp.float32),
            pltpu.SemaphoreType.DMA,
            pltpu.SemaphoreType.DMA,
        ],
    )
    def _sc_degree(dst_hbm, out_hbm, acc_sh, dst_v, ones_v, stage_v,
                   sem_i, sem_s):
        c = lax.axis_index("c")
        s = lax.axis_index("s")

        @pl.when(s == 0)
        def _zero():
            def zbody(i, carry):
                stage_v[pl.ds(i * 16, 16)] = jnp.zeros((16,), jnp.float32)
                return carry
            lax.fori_loop(0, N_NODES // 16, zbody, 0)
            pltpu.sync_copy(stage_v, acc_sh)

        for k in range(DCHUNK // 16):
            ones_v[pl.ds(k * 16, 16)] = jnp.ones((16,), jnp.float32)
        plsc.subcore_barrier()

        base = c * EDGES_PER_SC + s * EDGES_PER_TILE

        def group_body(g, carry):
            descs = []
            for j in range(DK):
                off = pl.multiple_of(base + (g * DK + j) * DCHUNK, 8)
                descs.append(pltpu.async_copy(
                    dst_hbm.at[pl.ds(off, DCHUNK)], dst_v.at[j], sem_i))
            for d in descs:
                d.wait()
            sds = [pltpu.async_copy(ones_v, acc_sh.at[dst_v.at[j]], sem_s,
                                    add=True)
                   for j in range(DK)]
            for d in sds:
                d.wait()
            return carry

        lax.fori_loop(0, DN_GROUPS, group_body, 0)
        for t in range(DTAIL0, DN_CHUNKS):
            off = pl.multiple_of(base + t * DCHUNK, 8)
            pltpu.async_copy(dst_hbm.at[pl.ds(off, DCHUNK)], dst_v.at[0],
                             sem_i).wait()
            pltpu.async_copy(ones_v, acc_sh.at[dst_v.at[0]], sem_s,
                             add=True).wait()
        plsc.subcore_barrier()

        @pl.when(s == 0)
        def _copy_out():
            pltpu.sync_copy(acc_sh, stage_v)
            pltpu.sync_copy(
                stage_v,
                out_hbm.at[pl.ds(pl.multiple_of(c * N_NODES, 8), N_NODES)])

    @functools.partial(
        pl.kernel,
        mesh=mesh,
        out_type=jax.ShapeDtypeStruct((NC * N_NODES, F_HID), jnp.float32),
        scratch_types=[
            pltpu.VMEM_SHARED((N_NODES, F_HID), jnp.float32),
            pltpu.VMEM((SLOTS, CHUNK), jnp.int32),
            pltpu.VMEM((SLOTS, CHUNK), jnp.int32),
            pltpu.VMEM((SLOTS, CHUNK, F_HID), jnp.float32),
            pltpu.SemaphoreType.DMA,
            pltpu.SemaphoreType.DMA,
            pltpu.SemaphoreType.DMA,
        ],
    )
    def _sc_scatter(z_hbm, src_hbm, dst_hbm, zeros_hbm, out_hbm,
                    acc_sh, src_v, dst_v, rows_v, sem_i, sem_g, sem_s):
        c = lax.axis_index("c")
        s = lax.axis_index("s")
        row0 = pl.multiple_of(s * ZROWS, 8)
        base = c * EDGES_PER_SC + s * EDGES_PER_TILE

        def fire_idx(half, cbase):
            for j in range(HALF):
                off = pl.multiple_of(base + (cbase + j) * CHUNK, 8)
                slot = half * HALF + j
                pltpu.async_copy(src_hbm.at[pl.ds(off, CHUNK)], src_v.at[slot],
                                 sem_i)
                pltpu.async_copy(dst_hbm.at[pl.ds(off, CHUNK)], dst_v.at[slot],
                                 sem_i)

        def wait_idx(half):
            for j in range(HALF):
                slot = half * HALF + j
                pltpu.make_async_copy(src_hbm.at[pl.ds(0, CHUNK)],
                                      src_v.at[slot], sem_i).wait()
                pltpu.make_async_copy(src_hbm.at[pl.ds(0, CHUNK)],
                                      dst_v.at[slot], sem_i).wait()

        def fire_gather(half):
            for j in range(HALF):
                slot = half * HALF + j
                pltpu.async_copy(z_hbm.at[src_v.at[slot]], rows_v.at[slot],
                                 sem_g)

        def fire_scatter(half):
            for j in range(HALF):
                slot = half * HALF + j
                pltpu.async_copy(rows_v.at[slot], acc_sh.at[dst_v.at[slot]],
                                 sem_s, add=True)

        def wait_rows(half, sem):
            for j in range(HALF):
                slot = half * HALF + j
                pltpu.make_async_copy(z_hbm.at[pl.ds(0, CHUNK)],
                                      rows_v.at[slot], sem).wait()

        fire_idx(0, 0)
        pltpu.sync_copy(zeros_hbm, acc_sh.at[pl.ds(row0, ZROWS)])

        @pl.when(s == 0)
        def _zero_rem():
            pltpu.sync_copy(zeros_hbm.at[pl.ds(0, ZREM)],
                            acc_sh.at[pl.ds(NS * ZROWS, ZREM)])

        wait_idx(0)
        fire_gather(0)
        fire_idx(1, HALF)
        plsc.subcore_barrier()
        wait_rows(0, sem_g)
        fire_scatter(0)
        wait_idx(1)
        fire_gather(1)
        wait_rows(0, sem_s)
        fire_idx(0, SLOTS)
        wait_rows(1, sem_g)
        fire_scatter(1)
        wait_idx(0)
        fire_gather(0)

        def steady_body(i, carry):
            c0 = i * SLOTS
            wait_rows(1, sem_s)
            fire_idx(1, c0 + HALF)
            wait_rows(0, sem_g)
            fire_scatter(0)
            wait_idx(1)
            fire_gather(1)
            wait_rows(0, sem_s)

            @pl.when(i < N_ITER - 1)
            def _prefetch_idx():
                fire_idx(0, c0 + SLOTS)

            wait_rows(1, sem_g)
            fire_scatter(1)

            @pl.when(i < N_ITER - 1)
            def _prefetch_gather():
                wait_idx(0)
                fire_gather(0)

            return carry

        lax.fori_loop(1, N_ITER, steady_body, 0)
        wait_rows(1, sem_s)

        for t in range(TAIL0, N_CHUNKS):
            off = pl.multiple_of(base + t * CHUNK, 8)
            pltpu.async_copy(src_hbm.at[pl.ds(off, CHUNK)], src_v.at[0],
                             sem_i).wait()
            pltpu.async_copy(dst_hbm.at[pl.ds(off, CHUNK)], dst_v.at[0],
                             sem_i).wait()
            pltpu.async_copy(z_hbm.at[src_v.at[0]], rows_v.at[0], sem_g).wait()
            pltpu.async_copy(rows_v.at[0], acc_sh.at[dst_v.at[0]], sem_s,
                             add=True).wait()
        plsc.subcore_barrier()

        out_off = pl.multiple_of(c * N_NODES + row0, 8)
        pltpu.sync_copy(acc_sh.at[pl.ds(row0, ZROWS)],
                        out_hbm.at[pl.ds(out_off, ZROWS)])

        @pl.when(s == 0)
        def _out_rem():
            rem0 = NS * ZROWS
            pltpu.sync_copy(
                acc_sh.at[pl.ds(rem0, ZREM)],
                out_hbm.at[pl.ds(pl.multiple_of(c * N_NODES + rem0, 8), ZREM)])

    return _sc_degree, _sc_scatter


BLK = 1000


def _tc_first_body(d0, d1, x, w, dinv_ref, z_ref):
    dinv = lax.rsqrt(d0[...] + d1[...] + 1.0)
    dinv_ref[...] = dinv
    z_ref[...] = dinv * jnp.dot(x[...], w[...], preferred_element_type=jnp.float32)


def _tc_mid_body(a0, a1, z, dinv, b, w, zn_ref):
    h = jnp.maximum(dinv[...] * (a0[...] + a1[...] + z[...]) + b[...], 0.0)
    zn_ref[...] = dinv[...] * jnp.dot(h, w[...], preferred_element_type=jnp.float32)


def _tc_final_body(a0, a1, z, dinv, b3, wl, bl, out_ref):
    x3 = dinv[...] * (a0[...] + a1[...] + z[...]) + b3[...]
    logits = jnp.dot(x3, wl[...], preferred_element_type=jnp.float32) + bl[...]
    m = jnp.max(logits, axis=1, keepdims=True)
    lse = jnp.log(jnp.sum(jnp.exp(logits - m), axis=1, keepdims=True))
    out_ref[...] = (logits - m) - lse


def _rows(i):
    return (i, 0)


def _whole(i):
    return (0, 0)


_GRID = N_NODES // BLK

_tc_first = pl.pallas_call(
    _tc_first_body,
    grid=(_GRID,),
    in_specs=[
        pl.BlockSpec((BLK, 1), _rows),
        pl.BlockSpec((BLK, 1), _rows),
        pl.BlockSpec((BLK, F_HID), _rows),
        pl.BlockSpec((F_HID, F_HID), _whole),
    ],
    out_specs=[
        pl.BlockSpec((BLK, 1), _rows),
        pl.BlockSpec((BLK, F_HID), _rows),
    ],
    out_shape=[
        jax.ShapeDtypeStruct((N_NODES, 1), jnp.float32),
        jax.ShapeDtypeStruct((N_NODES, F_HID), jnp.float32),
    ],
)

_tc_mid = pl.pallas_call(
    _tc_mid_body,
    grid=(_GRID,),
    in_specs=[
        pl.BlockSpec((BLK, F_HID), _rows),
        pl.BlockSpec((BLK, F_HID), _rows),
        pl.BlockSpec((BLK, F_HID), _rows),
        pl.BlockSpec((BLK, 1), _rows),
        pl.BlockSpec((1, F_HID), _whole),
        pl.BlockSpec((F_HID, F_HID), _whole),
    ],
    out_specs=pl.BlockSpec((BLK, F_HID), _rows),
    out_shape=jax.ShapeDtypeStruct((N_NODES, F_HID), jnp.float32),
)

_tc_final = pl.pallas_call(
    _tc_final_body,
    grid=(_GRID,),
    in_specs=[
        pl.BlockSpec((BLK, F_HID), _rows),
        pl.BlockSpec((BLK, F_HID), _rows),
        pl.BlockSpec((BLK, F_HID), _rows),
        pl.BlockSpec((BLK, 1), _rows),
        pl.BlockSpec((1, F_HID), _whole),
        pl.BlockSpec((F_HID, N_CLS), _whole),
        pl.BlockSpec((1, N_CLS), _whole),
    ],
    out_specs=pl.BlockSpec((BLK, N_CLS), _rows),
    out_shape=jax.ShapeDtypeStruct((N_NODES, N_CLS), jnp.float32),
)


def kernel(x, edge_index, W1, b1, W2, b2, W3, b3, Wl, bl):
    src = edge_index[0]
    dst = edge_index[1]
    zeros_blk = jnp.zeros((ZROWS, F_HID), jnp.float32)

    _sc_degree, _sc_scatter = _sc_kernels()
    deg = _sc_degree(dst).reshape(NC, N_NODES)
    d0 = deg[0].reshape(N_NODES, 1)
    d1 = deg[1].reshape(N_NODES, 1)

    dinv, z1 = _tc_first(d0, d1, x, W1)
    acc = _sc_scatter(z1, src, dst, zeros_blk).reshape(NC, N_NODES, F_HID)
    z2 = _tc_mid(acc[0], acc[1], z1, dinv, b1.reshape(1, F_HID), W2)
    acc = _sc_scatter(z2, src, dst, zeros_blk).reshape(NC, N_NODES, F_HID)
    z3 = _tc_mid(acc[0], acc[1], z2, dinv, b2.reshape(1, F_HID), W3)
    acc = _sc_scatter(z3, src, dst, zeros_blk).reshape(NC, N_NODES, F_HID)
    return _tc_final(acc[0], acc[1], z3, dinv, b3.reshape(1, F_HID),
                     Wl, bl.reshape(1, N_CLS))


# speedup vs baseline: 1.0556x; 1.0110x over previous
import functools

import jax
import jax.numpy as jnp
from jax import lax
from jax.experimental import pallas as pl
from jax.experimental.pallas import tpu as pltpu
from jax.experimental.pallas import tpu_sc as plsc

N_NODES = 10000
N_EDGES = 320000
F_HID = 128
N_CLS = 64

NC = 2
NS = 16
EDGES_PER_SC = N_EDGES // NC
EDGES_PER_TILE = EDGES_PER_SC // NS
CHUNK = 80
HALF = 2
SLOTS = 2 * HALF
N_CHUNKS = EDGES_PER_TILE // CHUNK
N_ITER = N_CHUNKS // SLOTS
TAIL0 = N_ITER * SLOTS

DCHUNK = 80
DN_CHUNKS = EDGES_PER_TILE // DCHUNK
DK = 8
DN_GROUPS = DN_CHUNKS // DK
DTAIL0 = DN_GROUPS * DK
ZROWS = 624
ZREM = N_NODES - NS * ZROWS

@functools.cache
def _sc_kernels():
    mesh = plsc.VectorSubcoreMesh(
        core_axis_name="c", subcore_axis_name="s", num_cores=NC, num_subcores=NS
    )

    @functools.partial(
        pl.kernel,
        mesh=mesh,
        out_type=jax.ShapeDtypeStruct((NC * N_NODES,), jnp.float32),
        scratch_types=[
            pltpu.VMEM_SHARED((N_NODES,), jnp.float32),
            pltpu.VMEM((DK, DCHUNK), jnp.int32),
            pltpu.VMEM((DCHUNK,), jnp.float32),
            pltpu.VMEM((N_NODES,), jnp.float32),
            pltpu.SemaphoreType.DMA,
            pltpu.SemaphoreType.DMA,
        ],
    )
    def _sc_degree(dst_hbm, out_hbm, acc_sh, dst_v, ones_v, stage_v,
                   sem_i, sem_s):
        c = lax.axis_index("c")
        s = lax.axis_index("s")

        @pl.when(s == 0)
        def _zero():
            def zbody(i, carry):
                stage_v[pl.ds(i * 16, 16)] = jnp.zeros((16,), jnp.float32)
                return carry
            lax.fori_loop(0, N_NODES // 16, zbody, 0)
            pltpu.sync_copy(stage_v, acc_sh)

        for k in range(DCHUNK // 16):
            ones_v[pl.ds(k * 16, 16)] = jnp.ones((16,), jnp.float32)
        plsc.subcore_barrier()

        base = c * EDGES_PER_SC + s * EDGES_PER_TILE

        def group_body(g, carry):
            descs = []
            for j in range(DK):
                off = pl.multiple_of(base + (g * DK + j) * DCHUNK, 8)
                descs.append(pltpu.async_copy(
                    dst_hbm.at[pl.ds(off, DCHUNK)], dst_v.at[j], sem_i))
            for d in descs:
                d.wait()
            sds = [pltpu.async_copy(ones_v, acc_sh.at[dst_v.at[j]], sem_s,
                                    add=True)
                   for j in range(DK)]
            for d in sds:
                d.wait()
            return carry

        lax.fori_loop(0, DN_GROUPS, group_body, 0)
        for t in range(DTAIL0, DN_CHUNKS):
            off = pl.multiple_of(base + t * DCHUNK, 8)
            pltpu.async_copy(dst_hbm.at[pl.ds(off, DCHUNK)], dst_v.at[0],
                             sem_i).wait()
            pltpu.async_copy(ones_v, acc_sh.at[dst_v.at[0]], sem_s,
                             add=True).wait()
        plsc.subcore_barrier()

        @pl.when(s == 0)
        def _copy_out():
            pltpu.sync_copy(acc_sh, stage_v)
            pltpu.sync_copy(
                stage_v,
                out_hbm.at[pl.ds(pl.multiple_of(c * N_NODES, 8), N_NODES)])

    @functools.partial(
        pl.kernel,
        mesh=mesh,
        out_type=jax.ShapeDtypeStruct((NC * N_NODES, F_HID), jnp.float32),
        scratch_types=[
            pltpu.VMEM_SHARED((N_NODES, F_HID), jnp.float32),
            pltpu.VMEM((SLOTS, CHUNK), jnp.int32),
            pltpu.VMEM((SLOTS, CHUNK), jnp.int32),
            pltpu.VMEM((SLOTS, CHUNK, F_HID), jnp.float32),
            pltpu.SemaphoreType.DMA,
            pltpu.SemaphoreType.DMA,
            pltpu.SemaphoreType.DMA,
        ],
    )
    def _sc_scatter(z_hbm, src_hbm, dst_hbm, zeros_hbm, out_hbm,
                    acc_sh, src_v, dst_v, rows_v, sem_i, sem_g, sem_s):
        c = lax.axis_index("c")
        s = lax.axis_index("s")
        row0 = pl.multiple_of(s * ZROWS, 8)
        base = c * EDGES_PER_SC + s * EDGES_PER_TILE

        def fire_idx(half, cbase):
            for j in range(HALF):
                off = pl.multiple_of(base + (cbase + j) * CHUNK, 8)
                slot = half * HALF + j
                pltpu.async_copy(src_hbm.at[pl.ds(off, CHUNK)], src_v.at[slot],
                                 sem_i)
                pltpu.async_copy(dst_hbm.at[pl.ds(off, CHUNK)], dst_v.at[slot],
                                 sem_i)

        def wait_idx(half):
            for j in range(HALF):
                slot = half * HALF + j
                pltpu.make_async_copy(src_hbm.at[pl.ds(0, CHUNK)],
                                      src_v.at[slot], sem_i).wait()
                pltpu.make_async_copy(src_hbm.at[pl.ds(0, CHUNK)],
                                      dst_v.at[slot], sem_i).wait()

        def fire_gather(half):
            for j in range(HALF):
                slot = half * HALF + j
                pltpu.async_copy(z_hbm.at[src_v.at[slot]], rows_v.at[slot],
                                 sem_g)

        def fire_scatter(half):
            for j in range(HALF):
                slot = half * HALF + j
                pltpu.async_copy(rows_v.at[slot], acc_sh.at[dst_v.at[slot]],
                                 sem_s, add=True)

        def wait_rows(half, sem):
            for j in range(HALF):
                slot = half * HALF + j
                pltpu.make_async_copy(z_hbm.at[pl.ds(0, CHUNK)],
                                      rows_v.at[slot], sem).wait()

        fire_idx(0, 0)
        pltpu.sync_copy(zeros_hbm, acc_sh.at[pl.ds(row0, ZROWS)])

        @pl.when(s == 0)
        def _zero_rem():
            pltpu.sync_copy(zeros_hbm.at[pl.ds(0, ZREM)],
                            acc_sh.at[pl.ds(NS * ZROWS, ZREM)])

        wait_idx(0)
        fire_gather(0)
        fire_idx(1, HALF)
        plsc.subcore_barrier()
        wait_rows(0, sem_g)
        fire_scatter(0)
        wait_idx(1)
        fire_gather(1)
        wait_rows(0, sem_s)
        fire_idx(0, SLOTS)
        wait_rows(1, sem_g)
        fire_scatter(1)
        wait_idx(0)
        fire_gather(0)

        def steady_body(i, carry):
            c0 = i * SLOTS
            wait_rows(1, sem_s)
            fire_idx(1, c0 + HALF)
            wait_rows(0, sem_g)
            fire_scatter(0)
            wait_idx(1)
            fire_gather(1)
            wait_rows(0, sem_s)

            @pl.when(i < N_ITER - 1)
            def _prefetch_idx():
                fire_idx(0, c0 + SLOTS)

            wait_rows(1, sem_g)
            fire_scatter(1)

            @pl.when(i < N_ITER - 1)
            def _prefetch_gather():
                wait_idx(0)
                fire_gather(0)

            return carry

        lax.fori_loop(1, N_ITER, steady_body, 0)
        wait_rows(1, sem_s)

        for t in range(TAIL0, N_CHUNKS):
            off = pl.multiple_of(base + t * CHUNK, 8)
            pltpu.async_copy(src_hbm.at[pl.ds(off, CHUNK)], src_v.at[0],
                             sem_i).wait()
            pltpu.async_copy(dst_hbm.at[pl.ds(off, CHUNK)], dst_v.at[0],
                             sem_i).wait()
            pltpu.async_copy(z_hbm.at[src_v.at[0]], rows_v.at[0], sem_g).wait()
            pltpu.async_copy(rows_v.at[0], acc_sh.at[dst_v.at[0]], sem_s,
                             add=True).wait()
        plsc.subcore_barrier()

        out_off = pl.multiple_of(c * N_NODES + row0, 8)
        pltpu.sync_copy(acc_sh.at[pl.ds(row0, ZROWS)],
                        out_hbm.at[pl.ds(out_off, ZROWS)])

        @pl.when(s == 0)
        def _out_rem():
            rem0 = NS * ZROWS
            pltpu.sync_copy(
                acc_sh.at[pl.ds(rem0, ZREM)],
                out_hbm.at[pl.ds(pl.multiple_of(c * N_NODES + rem0, 8), ZREM)])

    return _sc_degree, _sc_scatter


BLK = 1000


def _tc_first_body(d0, d1, x, w, dinv_ref, z_ref):
    dinv = lax.rsqrt(d0[...] + d1[...] + 1.0)
    dinv_ref[...] = dinv
    z_ref[...] = dinv * jnp.dot(x[...], w[...], preferred_element_type=jnp.float32)


def _tc_mid_body(a0, a1, z, dinv, b, w, zn_ref):
    h = jnp.maximum(dinv[...] * (a0[...] + a1[...] + z[...]) + b[...], 0.0)
    zn_ref[...] = dinv[...] * jnp.dot(h, w[...], preferred_element_type=jnp.float32)


def _tc_final_body(a0, a1, z, dinv, b3, wl, bl, out_ref):
    x3 = dinv[...] * (a0[...] + a1[...] + z[...]) + b3[...]
    logits = jnp.dot(x3, wl[...], preferred_element_type=jnp.float32) + bl[...]
    m = jnp.max(logits, axis=1, keepdims=True)
    lse = jnp.log(jnp.sum(jnp.exp(logits - m), axis=1, keepdims=True))
    out_ref[...] = (logits - m) - lse


def _rows(i):
    return (i, 0)


def _whole(i):
    return (0, 0)


_GRID = N_NODES // BLK

_tc_first = pl.pallas_call(
    _tc_first_body,
    grid=(_GRID,),
    in_specs=[
        pl.BlockSpec((BLK, 1), _rows),
        pl.BlockSpec((BLK, 1), _rows),
        pl.BlockSpec((BLK, F_HID), _rows),
        pl.BlockSpec((F_HID, F_HID), _whole),
    ],
    out_specs=[
        pl.BlockSpec((BLK, 1), _rows),
        pl.BlockSpec((BLK, F_HID), _rows),
    ],
    out_shape=[
        jax.ShapeDtypeStruct((N_NODES, 1), jnp.float32),
        jax.ShapeDtypeStruct((N_NODES, F_HID), jnp.float32),
    ],
)

_tc_mid = pl.pallas_call(
    _tc_mid_body,
    grid=(_GRID,),
    in_specs=[
        pl.BlockSpec((BLK, F_HID), _rows),
        pl.BlockSpec((BLK, F_HID), _rows),
        pl.BlockSpec((BLK, F_HID), _rows),
        pl.BlockSpec((BLK, 1), _rows),
        pl.BlockSpec((1, F_HID), _whole),
        pl.BlockSpec((F_HID, F_HID), _whole),
    ],
    out_specs=pl.BlockSpec((BLK, F_HID), _rows),
    out_shape=jax.ShapeDtypeStruct((N_NODES, F_HID), jnp.float32),
)

_tc_final = pl.pallas_call(
    _tc_final_body,
    grid=(_GRID,),
    in_specs=[
        pl.BlockSpec((BLK, F_HID), _rows),
        pl.BlockSpec((BLK, F_HID), _rows),
        pl.BlockSpec((BLK, F_HID), _rows),
        pl.BlockSpec((BLK, 1), _rows),
        pl.BlockSpec((1, F_HID), _whole),
        pl.BlockSpec((F_HID, N_CLS), _whole),
        pl.BlockSpec((1, N_CLS), _whole),
    ],
    out_specs=pl.BlockSpec((BLK, N_CLS), _rows),
    out_shape=jax.ShapeDtypeStruct((N_NODES, N_CLS), jnp.float32),
)


def kernel(x, edge_index, W1, b1, W2, b2, W3, b3, Wl, bl):
    src = edge_index[0]
    dst = edge_index[1]
    zeros_blk = jnp.zeros((ZROWS, F_HID), jnp.float32)

    _sc_degree, _sc_scatter = _sc_kernels()
    deg = _sc_degree(dst).reshape(NC, N_NODES)
    d0 = deg[0].reshape(N_NODES, 1)
    d1 = deg[1].reshape(N_NODES, 1)

    dinv, z1 = _tc_first(d0, d1, x, W1)
    acc = _sc_scatter(z1, src, dst, zeros_blk).reshape(NC, N_NODES, F_HID)
    z2 = _tc_mid(acc[0], acc[1], z1, dinv, b1.reshape(1, F_HID), W2)
    acc = _sc_scatter(z2, src, dst, zeros_blk).reshape(NC, N_NODES, F_HID)
    z3 = _tc_mid(acc[0], acc[1], z2, dinv, b2.reshape(1, F_HID), W3)
    acc = _sc_scatter(z3, src, dst, zeros_blk).reshape(NC, N_NODES, F_HID)
    return _tc_final(acc[0], acc[1], z3, dinv, b3.reshape(1, F_HID),
                     Wl, bl.reshape(1, N_CLS))


# speedup vs baseline: 1.0570x; 1.0014x over previous
import functools

import jax
import jax.numpy as jnp
from jax import lax
from jax.experimental import pallas as pl
from jax.experimental.pallas import tpu as pltpu
from jax.experimental.pallas import tpu_sc as plsc

N_NODES = 10000
N_EDGES = 320000
F_HID = 128
N_CLS = 64

NC = 2
NS = 16
EDGES_PER_SC = N_EDGES // NC
EDGES_PER_TILE = EDGES_PER_SC // NS
CHUNK = 80
HALF = 2
SLOTS = 2 * HALF
N_CHUNKS = EDGES_PER_TILE // CHUNK
N_ITER = N_CHUNKS // SLOTS
TAIL0 = N_ITER * SLOTS

DCHUNK = 80
DN_CHUNKS = EDGES_PER_TILE // DCHUNK
DK = 8
DN_GROUPS = DN_CHUNKS // DK
DTAIL0 = DN_GROUPS * DK
ZROWS = 624
ZREM = N_NODES - NS * ZROWS

@functools.cache
def _sc_kernels():
    mesh = plsc.VectorSubcoreMesh(
        core_axis_name="c", subcore_axis_name="s", num_cores=NC, num_subcores=NS
    )

    @functools.partial(
        pl.kernel,
        mesh=mesh,
        out_type=jax.ShapeDtypeStruct((NC * N_NODES,), jnp.float32),
        scratch_types=[
            pltpu.VMEM_SHARED((N_NODES,), jnp.float32),
            pltpu.VMEM((DK, DCHUNK), jnp.int32),
            pltpu.VMEM((DCHUNK,), jnp.float32),
            pltpu.VMEM((N_NODES,), jnp.float32),
            pltpu.SemaphoreType.DMA,
            pltpu.SemaphoreType.DMA,
        ],
    )
    def _sc_degree(dst_hbm, out_hbm, acc_sh, dst_v, ones_v, stage_v,
                   sem_i, sem_s):
        c = lax.axis_index("c")
        s = lax.axis_index("s")

        @pl.when(s == 0)
        def _zero():
            def zbody(i, carry):
                stage_v[pl.ds(i * 16, 16)] = jnp.zeros((16,), jnp.float32)
                return carry
            lax.fori_loop(0, N_NODES // 16, zbody, 0)
            pltpu.sync_copy(stage_v, acc_sh)

        for k in range(DCHUNK // 16):
            ones_v[pl.ds(k * 16, 16)] = jnp.ones((16,), jnp.float32)
        plsc.subcore_barrier()

        base = c * EDGES_PER_SC + s * EDGES_PER_TILE

        def group_body(g, carry):
            descs = []
            for j in range(DK):
                off = pl.multiple_of(base + (g * DK + j) * DCHUNK, 8)
                descs.append(pltpu.async_copy(
                    dst_hbm.at[pl.ds(off, DCHUNK)], dst_v.at[j], sem_i))
            for d in descs:
                d.wait()
            sds = [pltpu.async_copy(ones_v, acc_sh.at[dst_v.at[j]], sem_s,
                                    add=True)
                   for j in range(DK)]
            for d in sds:
                d.wait()
            return carry

        lax.fori_loop(0, DN_GROUPS, group_body, 0)
        for t in range(DTAIL0, DN_CHUNKS):
            off = pl.multiple_of(base + t * DCHUNK, 8)
            pltpu.async_copy(dst_hbm.at[pl.ds(off, DCHUNK)], dst_v.at[0],
                             sem_i).wait()
            pltpu.async_copy(ones_v, acc_sh.at[dst_v.at[0]], sem_s,
                             add=True).wait()
        plsc.subcore_barrier()

        @pl.when(s == 0)
        def _copy_out():
            pltpu.sync_copy(acc_sh, stage_v)
            pltpu.sync_copy(
                stage_v,
                out_hbm.at[pl.ds(pl.multiple_of(c * N_NODES, 8), N_NODES)])

    @functools.partial(
        pl.kernel,
        mesh=mesh,
        out_type=jax.ShapeDtypeStruct((NC * N_NODES, F_HID), jnp.float32),
        scratch_types=[
            pltpu.VMEM_SHARED((N_NODES, F_HID), jnp.float32),
            pltpu.VMEM((SLOTS, CHUNK), jnp.int32),
            pltpu.VMEM((SLOTS, CHUNK), jnp.int32),
            pltpu.VMEM((SLOTS, CHUNK, F_HID), jnp.float32),
            pltpu.SemaphoreType.DMA,
            pltpu.SemaphoreType.DMA,
            pltpu.SemaphoreType.DMA,
        ],
    )
    def _sc_scatter(z_hbm, src_hbm, dst_hbm, zeros_hbm, out_hbm,
                    acc_sh, src_v, dst_v, rows_v, sem_i, sem_g, sem_s):
        c = lax.axis_index("c")
        s = lax.axis_index("s")
        row0 = pl.multiple_of(s * ZROWS, 8)
        base = c * EDGES_PER_SC + s * EDGES_PER_TILE

        def fire_idx(half, cbase):
            for j in range(HALF):
                off = pl.multiple_of(base + (cbase + j) * CHUNK, 8)
                slot = half * HALF + j
                pltpu.async_copy(src_hbm.at[pl.ds(off, CHUNK)], src_v.at[slot],
                                 sem_i)
                pltpu.async_copy(dst_hbm.at[pl.ds(off, CHUNK)], dst_v.at[slot],
                                 sem_i)

        def wait_idx(half):
            for j in range(HALF):
                slot = half * HALF + j
                pltpu.make_async_copy(src_hbm.at[pl.ds(0, CHUNK)],
                                      src_v.at[slot], sem_i).wait()
                pltpu.make_async_copy(src_hbm.at[pl.ds(0, CHUNK)],
                                      dst_v.at[slot], sem_i).wait()

        def fire_gather(half):
            for j in range(HALF):
                slot = half * HALF + j
                pltpu.async_copy(z_hbm.at[src_v.at[slot]], rows_v.at[slot],
                                 sem_g)

        def fire_scatter(half):
            for j in range(HALF):
                slot = half * HALF + j
                pltpu.async_copy(rows_v.at[slot], acc_sh.at[dst_v.at[slot]],
                                 sem_s, add=True)

        def wait_rows(half, sem):
            for j in range(HALF):
                slot = half * HALF + j
                pltpu.make_async_copy(z_hbm.at[pl.ds(0, CHUNK)],
                                      rows_v.at[slot], sem).wait()

        fire_idx(0, 0)
        pltpu.sync_copy(zeros_hbm, acc_sh.at[pl.ds(row0, ZROWS)])

        @pl.when(s == 0)
        def _zero_rem():
            pltpu.sync_copy(zeros_hbm.at[pl.ds(0, ZREM)],
                            acc_sh.at[pl.ds(NS * ZROWS, ZREM)])

        wait_idx(0)
        fire_gather(0)
        fire_idx(1, HALF)
        plsc.subcore_barrier()
        wait_rows(0, sem_g)
        fire_scatter(0)
        wait_idx(1)
        fire_gather(1)
        wait_rows(0, sem_s)
        fire_idx(0, SLOTS)
        wait_rows(1, sem_g)
        fire_scatter(1)
        wait_idx(0)
        fire_gather(0)

        def steady_body(i, carry):
            c0 = i * SLOTS
            wait_rows(1, sem_s)
            fire_idx(1, c0 + HALF)
            wait_rows(0, sem_g)
            fire_scatter(0)
            wait_idx(1)
            fire_gather(1)
            wait_rows(0, sem_s)

            @pl.when(i < N_ITER - 1)
            def _prefetch_idx():
                fire_idx(0, c0 + SLOTS)

            wait_rows(1, sem_g)
            fire_scatter(1)

            @pl.when(i < N_ITER - 1)
            def _prefetch_gather():
                wait_idx(0)
                fire_gather(0)

            return carry

        lax.fori_loop(1, N_ITER, steady_body, 0)
        wait_rows(1, sem_s)

        for t in range(TAIL0, N_CHUNKS):
            off = pl.multiple_of(base + t * CHUNK, 8)
            pltpu.async_copy(src_hbm.at[pl.ds(off, CHUNK)], src_v.at[0],
                             sem_i).wait()
            pltpu.async_copy(dst_hbm.at[pl.ds(off, CHUNK)], dst_v.at[0],
                             sem_i).wait()
            pltpu.async_copy(z_hbm.at[src_v.at[0]], rows_v.at[0], sem_g).wait()
            pltpu.async_copy(rows_v.at[0], acc_sh.at[dst_v.at[0]], sem_s,
                             add=True).wait()
        plsc.subcore_barrier()

        out_off = pl.multiple_of(c * N_NODES + row0, 8)
        pltpu.sync_copy(acc_sh.at[pl.ds(row0, ZROWS)],
                        out_hbm.at[pl.ds(out_off, ZROWS)])

        @pl.when(s == 0)
        def _out_rem():
            rem0 = NS * ZROWS
            pltpu.sync_copy(
                acc_sh.at[pl.ds(rem0, ZREM)],
                out_hbm.at[pl.ds(pl.multiple_of(c * N_NODES + rem0, 8), ZREM)])

    return _sc_degree, _sc_scatter


BLK = 1000


def _tc_xw_body(x, w, xw_ref):
    xw_ref[...] = jnp.dot(x[...], w[...], preferred_element_type=jnp.float32)


def _tc_scale_body(d0, d1, xw, dinv_ref, z_ref):
    dinv = lax.rsqrt(d0[...] + d1[...] + 1.0)
    dinv_ref[...] = dinv
    z_ref[...] = dinv * xw[...]


def _tc_mid_body(a0, a1, z, dinv, b, w, zn_ref):
    h = jnp.maximum(dinv[...] * (a0[...] + a1[...] + z[...]) + b[...], 0.0)
    zn_ref[...] = dinv[...] * jnp.dot(h, w[...], preferred_element_type=jnp.float32)


def _tc_final_body(a0, a1, z, dinv, b3, wl, bl, out_ref):
    x3 = dinv[...] * (a0[...] + a1[...] + z[...]) + b3[...]
    logits = jnp.dot(x3, wl[...], preferred_element_type=jnp.float32) + bl[...]
    m = jnp.max(logits, axis=1, keepdims=True)
    lse = jnp.log(jnp.sum(jnp.exp(logits - m), axis=1, keepdims=True))
    out_ref[...] = (logits - m) - lse


def _rows(i):
    return (i, 0)


def _whole(i):
    return (0, 0)


_GRID = N_NODES // BLK

_tc_xw = pl.pallas_call(
    _tc_xw_body,
    grid=(_GRID,),
    in_specs=[
        pl.BlockSpec((BLK, F_HID), _rows),
        pl.BlockSpec((F_HID, F_HID), _whole),
    ],
    out_specs=pl.BlockSpec((BLK, F_HID), _rows),
    out_shape=jax.ShapeDtypeStruct((N_NODES, F_HID), jnp.float32),
)

_tc_scale = pl.pallas_call(
    _tc_scale_body,
    grid=(_GRID,),
    in_specs=[
        pl.BlockSpec((BLK, 1), _rows),
        pl.BlockSpec((BLK, 1), _rows),
        pl.BlockSpec((BLK, F_HID), _rows),
    ],
    out_specs=[
        pl.BlockSpec((BLK, 1), _rows),
        pl.BlockSpec((BLK, F_HID), _rows),
    ],
    out_shape=[
        jax.ShapeDtypeStruct((N_NODES, 1), jnp.float32),
        jax.ShapeDtypeStruct((N_NODES, F_HID), jnp.float32),
    ],
)

_tc_mid = pl.pallas_call(
    _tc_mid_body,
    grid=(_GRID,),
    in_specs=[
        pl.BlockSpec((BLK, F_HID), _rows),
        pl.BlockSpec((BLK, F_HID), _rows),
        pl.BlockSpec((BLK, F_HID), _rows),
        pl.BlockSpec((BLK, 1), _rows),
        pl.BlockSpec((1, F_HID), _whole),
        pl.BlockSpec((F_HID, F_HID), _whole),
    ],
    out_specs=pl.BlockSpec((BLK, F_HID), _rows),
    out_shape=jax.ShapeDtypeStruct((N_NODES, F_HID), jnp.float32),
)

_tc_final = pl.pallas_call(
    _tc_final_body,
    grid=(_GRID,),
    in_specs=[
        pl.BlockSpec((BLK, F_HID), _rows),
        pl.BlockSpec((BLK, F_HID), _rows),
        pl.BlockSpec((BLK, F_HID), _rows),
        pl.BlockSpec((BLK, 1), _rows),
        pl.BlockSpec((1, F_HID), _whole),
        pl.BlockSpec((F_HID, N_CLS), _whole),
        pl.BlockSpec((1, N_CLS), _whole),
    ],
    out_specs=pl.BlockSpec((BLK, N_CLS), _rows),
    out_shape=jax.ShapeDtypeStruct((N_NODES, N_CLS), jnp.float32),
)


def kernel(x, edge_index, W1, b1, W2, b2, W3, b3, Wl, bl):
    src = edge_index[0]
    dst = edge_index[1]
    zeros_blk = jnp.zeros((ZROWS, F_HID), jnp.float32)

    _sc_degree, _sc_scatter = _sc_kernels()
    deg = _sc_degree(dst).reshape(NC, N_NODES)
    xw1 = _tc_xw(x, W1)
    d0 = deg[0].reshape(N_NODES, 1)
    d1 = deg[1].reshape(N_NODES, 1)

    dinv, z1 = _tc_scale(d0, d1, xw1)
    acc = _sc_scatter(z1, src, dst, zeros_blk).reshape(NC, N_NODES, F_HID)
    z2 = _tc_mid(acc[0], acc[1], z1, dinv, b1.reshape(1, F_HID), W2)
    acc = _sc_scatter(z2, src, dst, zeros_blk).reshape(NC, N_NODES, F_HID)
    z3 = _tc_mid(acc[0], acc[1], z2, dinv, b2.reshape(1, F_HID), W3)
    acc = _sc_scatter(z3, src, dst, zeros_blk).reshape(NC, N_NODES, F_HID)
    return _tc_final(acc[0], acc[1], z3, dinv, b3.reshape(1, F_HID),
                     Wl, bl.reshape(1, N_CLS))
